# baseline (speedup 1.0000x reference)
import functools

import numpy as np
import jax
import jax.numpy as jnp
from jax import lax
from jax.experimental import pallas as pl
from jax.experimental.pallas import tpu as pltpu
from jax.experimental.pallas import tpu_sc as plsc

N_SPH = 7
K_RAD = 6
NK = N_SPH * K_RAD
CUTOFF = 5.0


_NC = 2
_NS = 16
_NW = _NC * _NS
_CHUNK = 128


def _sc_gather_body(dist_hbm, idx_hbm, out_hbm, idx_v, rows_v, sem):
    wid = lax.axis_index("s") * _NC + lax.axis_index("c")
    n_chunks = idx_v.shape[0]
    pltpu.sync_copy(idx_hbm.at[wid], idx_v)

    def fire(c, _):
        pltpu.async_copy(dist_hbm.at[idx_v.at[c]], rows_v.at[c], sem)
        return _

    lax.fori_loop(0, n_chunks, fire, 0, unroll=False)

    def drain(c, _):
        pltpu.make_async_copy(dist_hbm.at[idx_v.at[c]], rows_v.at[c], sem).wait()
        return _

    lax.fori_loop(0, n_chunks, drain, 0, unroll=False)
    pltpu.sync_copy(rows_v, out_hbm.at[wid])


def _sc_gather(dist, idx):
    t = idx.shape[0]
    per_w_chunks = -(-t // (_NW * _CHUNK))
    t_pad = _NW * per_w_chunks * _CHUNK
    idx = jnp.pad(idx.astype(jnp.int32), (0, t_pad - t))
    idx3 = idx.reshape(_NW, per_w_chunks, _CHUNK)

    mesh = plsc.VectorSubcoreMesh(core_axis_name="c", subcore_axis_name="s")
    kern = functools.partial(
        pl.kernel,
        mesh=mesh,
        out_type=jax.ShapeDtypeStruct((_NW, per_w_chunks, _CHUNK), jnp.float32),
        scratch_types=[
            pltpu.VMEM((per_w_chunks, _CHUNK), jnp.int32),
            pltpu.VMEM((per_w_chunks, _CHUNK), jnp.float32),
            pltpu.SemaphoreType.DMA,
        ],
    )(_sc_gather_body)
    out = kern(dist, idx3)
    return out.reshape(t_pad)[:t]


_BT = 16000

_TWO_OVER_PI = float(np.float32(2.0 / np.pi))
_P2 = np.float64(np.pi / 2)
_P2_HI = np.frombuffer(
    (np.frombuffer(np.float32(_P2).tobytes(), np.uint32) & np.uint32(0xFFFFF000)).tobytes(),
    np.float32,
)[0]
_P2_MID = np.float32(_P2 - np.float64(_P2_HI))


def _sincos(u):
    kf = jnp.floor(u * _TWO_OVER_PI + 0.5)
    r = (u - kf * float(_P2_HI)) - kf * float(_P2_MID)
    r2 = r * r
    sp = ((-1.9515296e-4 * r2 + 8.3321608e-3) * r2 + -1.6666654e-1) * r2 * r + r
    cp = (((2.4433157e-5 * r2 + -1.3887316e-3) * r2 + 4.1666645e-2) * r2 + -0.5) * r2 + 1.0
    ki = kf.astype(jnp.int32)
    odd = (ki & 1) == 1
    s_base = jnp.where(odd, cp, sp)
    c_base = jnp.where(odd, sp, cp)
    s = jnp.where((ki & 2) == 2, -s_base, s_base)
    c = jnp.where(((ki + 1) & 2) == 2, -c_base, c_base)
    return s, c


def _tc_body(d_ref, a_ref, z_ref, n_ref, o_ref):
    d = d_ref[0]
    x = d / CUTOFF
    z = z_ref[...]
    u = z * x
    s, co = _sincos(u)
    inv = 1.0 / u
    j0 = s * inv
    j1 = (j0 - co) * inv

    a = a_ref[0]
    c = jnp.cos(a)
    p_prev, p_cur = jnp.ones_like(c), c
    ps = [p_prev, p_cur]
    for l in range(2, N_SPH):
        p_prev, p_cur = p_cur, ((2.0 * l - 1.0) * c * p_cur - (l - 1.0) * p_prev) / l
        ps.append(p_cur)

    out_rows = []
    for i in range(N_SPH):
        sl = slice(K_RAD * i, K_RAD * (i + 1))
        if i == 0:
            b = j0[sl]
        elif i == 1:
            b = j1[sl]
        else:
            jm2, jm1, inv_i = j0[sl], j1[sl], inv[sl]
            for l in range(2, i + 1):
                jm2, jm1 = jm1, (2.0 * l - 1.0) * inv_i * jm1 - jm2
            b = jm1
        ci = float(np.sqrt((2.0 * i + 1.0) / (4.0 * np.pi)).astype(np.float32))
        cbf_i = jnp.broadcast_to(ci * ps[i], (K_RAD, d.shape[1]))
        out_rows.append(b * n_ref[sl] * cbf_i)

    o_ref[...] = jnp.concatenate(out_rows, axis=0).T


def _tc_compute(dg, angle, zeros, norms, interpret=False):
    t = dg.shape[0]
    nb = t // _BT
    dg3 = dg.reshape(nb, 1, _BT)
    ang3 = angle.reshape(nb, 1, _BT)
    zcol = zeros.reshape(NK, 1)
    ncol = norms.reshape(NK, 1)
    return pl.pallas_call(
        _tc_body,
        grid=(nb,),
        in_specs=[
            pl.BlockSpec((1, 1, _BT), lambda i: (i, 0, 0)),
            pl.BlockSpec((1, 1, _BT), lambda i: (i, 0, 0)),
            pl.BlockSpec((NK, 1), lambda i: (0, 0)),
            pl.BlockSpec((NK, 1), lambda i: (0, 0)),
        ],
        out_specs=pl.BlockSpec((_BT, NK), lambda i: (i, 0)),
        out_shape=jax.ShapeDtypeStruct((t, NK), jnp.float32),
        interpret=interpret,
    )(dg3, ang3, zcol, ncol)


def kernel(dist, angle, idx_kj, zeros, norms):
    dg = _sc_gather(dist, idx_kj)
    return _tc_compute(dg, angle, zeros.astype(jnp.float32), norms.astype(jnp.float32))


# speedup vs baseline: 1.0092x; 1.0092x over previous
import functools

import numpy as np
import jax
import jax.numpy as jnp
from jax import lax
from jax.experimental import pallas as pl
from jax.experimental.pallas import tpu as pltpu
from jax.experimental.pallas import tpu_sc as plsc

N_SPH = 7
K_RAD = 6
NK = N_SPH * K_RAD
CUTOFF = 5.0


_NC = 2
_NS = 16
_NW = _NC * _NS
_CHUNK = 128


def _sc_gather_body(dist_hbm, idx_hbm, out_hbm, idx_v, rows_v, sem):
    wid = lax.axis_index("s") * _NC + lax.axis_index("c")
    n_chunks = idx_v.shape[0]
    pltpu.sync_copy(idx_hbm.at[wid], idx_v)

    def fire(c, _):
        pltpu.async_copy(dist_hbm.at[idx_v.at[c]], rows_v.at[c], sem)
        return _

    lax.fori_loop(0, n_chunks, fire, 0, unroll=False)

    def drain(c, _):
        pltpu.make_async_copy(dist_hbm.at[idx_v.at[c]], rows_v.at[c], sem).wait()
        return _

    lax.fori_loop(0, n_chunks, drain, 0, unroll=False)
    pltpu.sync_copy(rows_v, out_hbm.at[wid])


def _sc_gather(dist, idx):
    t = idx.shape[0]
    per_w_chunks = -(-t // (_NW * _CHUNK))
    t_pad = _NW * per_w_chunks * _CHUNK
    idx = jnp.pad(idx.astype(jnp.int32), (0, t_pad - t))
    idx3 = idx.reshape(_NW, per_w_chunks, _CHUNK)

    mesh = plsc.VectorSubcoreMesh(core_axis_name="c", subcore_axis_name="s")
    kern = functools.partial(
        pl.kernel,
        mesh=mesh,
        out_type=jax.ShapeDtypeStruct((_NW, per_w_chunks, _CHUNK), jnp.float32),
        scratch_types=[
            pltpu.VMEM((per_w_chunks, _CHUNK), jnp.int32),
            pltpu.VMEM((per_w_chunks, _CHUNK), jnp.float32),
            pltpu.SemaphoreType.DMA,
        ],
    )(_sc_gather_body)
    out = kern(dist, idx3)
    return out.reshape(t_pad)[:t]


_BT = 16000

_TWO_OVER_PI = float(np.float32(2.0 / np.pi))
_P2 = np.float64(np.pi / 2)
_P2_HI = np.frombuffer(
    (np.frombuffer(np.float32(_P2).tobytes(), np.uint32) & np.uint32(0xFFFFF000)).tobytes(),
    np.float32,
)[0]
_P2_MID = np.float32(_P2 - np.float64(_P2_HI))


def _sincos(u):
    kf = jnp.floor(u * _TWO_OVER_PI + 0.5)
    r = (u - kf * float(_P2_HI)) - kf * float(_P2_MID)
    r2 = r * r
    sp = ((-1.9515296e-4 * r2 + 8.3321608e-3) * r2 + -1.6666654e-1) * r2 * r + r
    cp = (((2.4433157e-5 * r2 + -1.3887316e-3) * r2 + 4.1666645e-2) * r2 + -0.5) * r2 + 1.0
    ki = kf.astype(jnp.int32)
    odd = (ki & 1) == 1
    s_base = jnp.where(odd, cp, sp)
    c_base = jnp.where(odd, sp, cp)
    s = jnp.where((ki & 2) == 2, -s_base, s_base)
    c = jnp.where(((ki + 1) & 2) == 2, -c_base, c_base)
    return s, c


def _tc_body(d_ref, a_ref, z_ref, n_ref, o_ref):
    d = d_ref[0]
    x = d / CUTOFF
    z = z_ref[...]
    u = z * x
    s, co = _sincos(u)
    inv = 1.0 / u
    j0 = s * inv
    j1 = (j0 - co) * inv

    a = a_ref[0]
    c = _sincos(a)[1]
    p_prev, p_cur = jnp.ones_like(c), c
    ps = [p_prev, p_cur]
    for l in range(2, N_SPH):
        p_prev, p_cur = p_cur, ((2.0 * l - 1.0) * c * p_cur - (l - 1.0) * p_prev) / l
        ps.append(p_cur)

    out_rows = []
    for i in range(N_SPH):
        sl = slice(K_RAD * i, K_RAD * (i + 1))
        if i == 0:
            b = j0[sl]
        elif i == 1:
            b = j1[sl]
        else:
            jm2, jm1, inv_i = j0[sl], j1[sl], inv[sl]
            for l in range(2, i + 1):
                jm2, jm1 = jm1, (2.0 * l - 1.0) * inv_i * jm1 - jm2
            b = jm1
        ci = float(np.sqrt((2.0 * i + 1.0) / (4.0 * np.pi)).astype(np.float32))
        cbf_i = jnp.broadcast_to(ci * ps[i], (K_RAD, d.shape[1]))
        out_rows.append(b * n_ref[sl] * cbf_i)

    o_ref[...] = jnp.concatenate(out_rows, axis=0).T


def _tc_compute(dg, angle, zeros, norms, interpret=False):
    t = dg.shape[0]
    nb = t // _BT
    dg3 = dg.reshape(nb, 1, _BT)
    ang3 = angle.reshape(nb, 1, _BT)
    zcol = zeros.reshape(NK, 1)
    ncol = norms.reshape(NK, 1)
    return pl.pallas_call(
        _tc_body,
        grid=(nb,),
        in_specs=[
            pl.BlockSpec((1, 1, _BT), lambda i: (i, 0, 0)),
            pl.BlockSpec((1, 1, _BT), lambda i: (i, 0, 0)),
            pl.BlockSpec((NK, 1), lambda i: (0, 0)),
            pl.BlockSpec((NK, 1), lambda i: (0, 0)),
        ],
        out_specs=pl.BlockSpec((_BT, NK), lambda i: (i, 0)),
        out_shape=jax.ShapeDtypeStruct((t, NK), jnp.float32),
        interpret=interpret,
    )(dg3, ang3, zcol, ncol)


def kernel(dist, angle, idx_kj, zeros, norms):
    dg = _sc_gather(dist, idx_kj)
    return _tc_compute(dg, angle, zeros.astype(jnp.float32), norms.astype(jnp.float32))
